# baseline (speedup 1.0000x reference)
import functools

import jax
import jax.numpy as jnp
from jax import lax
from jax.experimental import pallas as pl
from jax.experimental.pallas import tpu as pltpu
from jax.experimental.pallas import tpu_sc as plsc

_B, _K, _D = 4096, 64, 128
_NC, _NS = 2, 16
_NW = _NC * _NS
_BPW = _B // _NW
_LANES = 16
_CHUNKS = _D // _LANES


def _select_body(pis_ref, g_ref, out_ref):
    x = pis_ref[...]
    g = g_ref[...]
    m = jnp.max(x, axis=1, keepdims=True)
    unnorm = jnp.exp(x - m)
    pis = unnorm / jnp.sum(unnorm, axis=1, keepdims=True)
    s = g + jnp.log(pis)
    smax = jnp.max(s, axis=1, keepdims=True)
    col = lax.broadcasted_iota(jnp.int32, s.shape, 1)
    row = lax.broadcasted_iota(jnp.int32, s.shape, 0)
    cand = jnp.where(s == smax, row * _K + col, jnp.int32(2**30))
    out_ref[...] = jnp.min(cand, axis=1)


def _select_flat_idx(z_pis, g):
    return pl.pallas_call(
        _select_body,
        out_shape=jax.ShapeDtypeStruct((_B,), jnp.int32),
    )(z_pis, g)


def _sc_gather_reparam(means_flat, lv_flat, flat_idx, eps):
    mesh = plsc.VectorSubcoreMesh(core_axis_name="c", subcore_axis_name="s")

    @functools.partial(
        pl.kernel,
        mesh=mesh,
        out_type=jax.ShapeDtypeStruct((_B, _D), jnp.float32),
        scratch_types=[
            pltpu.VMEM((_BPW,), jnp.int32),
            pltpu.VMEM((_BPW, _D), jnp.float32),
            pltpu.VMEM((_BPW, _D), jnp.float32),
            pltpu.VMEM((_BPW, _D), jnp.float32),
            pltpu.SemaphoreType.DMA,
        ],
    )
    def body(means_hbm, lv_hbm, idx_hbm, eps_hbm, out_hbm,
             idx_v, m_v, lv_v, e_v, sem):
        wid = lax.axis_index("s") * _NC + lax.axis_index("c")
        base = wid * _BPW
        pltpu.sync_copy(idx_hbm.at[pl.ds(base, _BPW)], idx_v)
        cp_m = pltpu.async_copy(means_hbm.at[idx_v], m_v, sem)
        cp_l = pltpu.async_copy(lv_hbm.at[idx_v], lv_v, sem)
        pltpu.sync_copy(eps_hbm.at[pl.ds(base, _BPW), :], e_v)
        cp_m.wait()
        cp_l.wait()

        def row_body(r, carry):
            for c in range(_CHUNKS):
                sl = pl.ds(c * _LANES, _LANES)
                m_v[r, sl] = m_v[r, sl] + jnp.exp(lv_v[r, sl] * 0.5) * e_v[r, sl]
            return carry

        lax.fori_loop(0, _BPW, row_body, 0)
        pltpu.sync_copy(m_v, out_hbm.at[pl.ds(base, _BPW), :])

    return body(means_flat, lv_flat, flat_idx, eps)


def kernel(z_means, z_log_vars, z_pis):
    g = jax.random.gumbel(jax.random.key(42), (_B, _K), jnp.float32)
    eps = jax.random.normal(jax.random.key(43), (_B, _D), jnp.float32)
    flat_idx = _select_flat_idx(z_pis, g)
    means_flat = z_means.reshape(_B * _K, _D)
    lv_flat = z_log_vars.reshape(_B * _K, _D)
    return _sc_gather_reparam(means_flat, lv_flat, flat_idx, eps)


# speedup vs baseline: 1.0018x; 1.0018x over previous
import functools

import jax
import jax.numpy as jnp
from jax import lax
from jax.experimental import pallas as pl
from jax.experimental.pallas import tpu as pltpu
from jax.experimental.pallas import tpu_sc as plsc

_B, _K, _D = 4096, 64, 128
_NC, _NS = 2, 16
_NW = _NC * _NS
_BPW = _B // _NW
_LANES = 16
_CHUNKS = _D // _LANES


def _select_body(pis_ref, g_ref, out_ref):
    x = pis_ref[...]
    g = g_ref[...]
    m = jnp.max(x, axis=1, keepdims=True)
    unnorm = jnp.exp(x - m)
    pis = unnorm / jnp.sum(unnorm, axis=1, keepdims=True)
    s = g + jnp.log(pis)
    smax = jnp.max(s, axis=1, keepdims=True)
    col = lax.broadcasted_iota(jnp.int32, s.shape, 1)
    row = lax.broadcasted_iota(jnp.int32, s.shape, 0)
    cand = jnp.where(s == smax, row * _K + col, jnp.int32(2**30))
    out_ref[...] = jnp.min(cand, axis=1)


def _select_flat_idx(z_pis, g):
    return pl.pallas_call(
        _select_body,
        out_shape=jax.ShapeDtypeStruct((_B,), jnp.int32),
    )(z_pis, g)


def _sc_gather_reparam(means_flat, lv_flat, flat_idx, eps):
    mesh = plsc.VectorSubcoreMesh(core_axis_name="c", subcore_axis_name="s")

    @functools.partial(
        pl.kernel,
        mesh=mesh,
        out_type=jax.ShapeDtypeStruct((_B, _D), jnp.float32),
        scratch_types=[
            pltpu.VMEM((_BPW,), jnp.int32),
            pltpu.VMEM((_BPW, _D), jnp.float32),
            pltpu.VMEM((_BPW, _D), jnp.float32),
            pltpu.VMEM((_BPW, _D), jnp.float32),
            pltpu.SemaphoreType.DMA,
        ],
    )
    def body(means_hbm, lv_hbm, idx_hbm, eps_hbm, out_hbm,
             idx_v, m_v, lv_v, e_v, sem):
        wid = lax.axis_index("s") * _NC + lax.axis_index("c")
        base = wid * _BPW
        pltpu.sync_copy(idx_hbm.at[pl.ds(base, _BPW)], idx_v)
        cp_m = pltpu.async_copy(means_hbm.at[idx_v], m_v, sem)
        cp_l = pltpu.async_copy(lv_hbm.at[idx_v], lv_v, sem)
        pltpu.sync_copy(eps_hbm.at[pl.ds(base, _BPW), :], e_v)
        cp_m.wait()
        cp_l.wait()

        def row_body(r, carry):
            for c in range(_CHUNKS):
                sl = pl.ds(c * _LANES, _LANES)
                m_v[r, sl] = m_v[r, sl] + jnp.exp(lv_v[r, sl] * 0.5) * e_v[r, sl]
            return carry

        lax.fori_loop(0, _BPW, row_body, 0)
        pltpu.sync_copy(m_v, out_hbm.at[pl.ds(base, _BPW), :])

    return body(means_flat, lv_flat, flat_idx, eps)


_NOISE_CACHE = []


def _noise():
    if not _NOISE_CACHE:
        def build():
            g = jax.random.gumbel(jax.random.key(42), (_B, _K), jnp.float32)
            eps = jax.random.normal(jax.random.key(43), (_B, _D), jnp.float32)
            return g, eps
        _NOISE_CACHE.append(jax.jit(build)())
    return _NOISE_CACHE[0]


def kernel(z_means, z_log_vars, z_pis):
    g, eps = _noise()
    flat_idx = _select_flat_idx(z_pis, g)
    means_flat = z_means.reshape(_B * _K, _D)
    lv_flat = z_log_vars.reshape(_B * _K, _D)
    return _sc_gather_reparam(means_flat, lv_flat, flat_idx, eps)


# speedup vs baseline: 1.2797x; 1.2774x over previous
import functools

import jax
import jax.numpy as jnp
from jax import lax
from jax.experimental import pallas as pl
from jax.experimental.pallas import tpu as pltpu
from jax.experimental.pallas import tpu_sc as plsc

_B, _K, _D = 4096, 64, 128
_NC, _NS = 2, 16
_NW = _NC * _NS
_BPW = _B // _NW
_LANES = 16
_CHUNKS = _D // _LANES


def _select_body(pis_ref, g_ref, out_ref):
    x = pis_ref[...]
    g = g_ref[...]
    m = jnp.max(x, axis=0, keepdims=True)
    unnorm = jnp.exp(x - m)
    pis = unnorm / jnp.sum(unnorm, axis=0, keepdims=True)
    s = g + jnp.log(pis)
    smax = jnp.max(s, axis=0, keepdims=True)
    kk = lax.broadcasted_iota(jnp.int32, s.shape, 0)
    bb = lax.broadcasted_iota(jnp.int32, s.shape, 1)
    cand = jnp.where(s == smax, bb * _K + kk, jnp.int32(2**30))
    out_ref[...] = jnp.min(cand, axis=0)


def _select_flat_idx(z_pis_t, g_t):
    return pl.pallas_call(
        _select_body,
        out_shape=jax.ShapeDtypeStruct((_B,), jnp.int32),
    )(z_pis_t, g_t)


def _sc_gather_reparam(means_flat, lv_flat, flat_idx, eps):
    mesh = plsc.VectorSubcoreMesh(core_axis_name="c", subcore_axis_name="s")

    @functools.partial(
        pl.kernel,
        mesh=mesh,
        out_type=jax.ShapeDtypeStruct((_B, _D), jnp.float32),
        scratch_types=[
            pltpu.VMEM((_BPW,), jnp.int32),
            pltpu.VMEM((_BPW, _D), jnp.float32),
            pltpu.VMEM((_BPW, _D), jnp.float32),
            pltpu.VMEM((_BPW, _D), jnp.float32),
            pltpu.SemaphoreType.DMA,
        ],
    )
    def body(means_hbm, lv_hbm, idx_hbm, eps_hbm, out_hbm,
             idx_v, m_v, lv_v, e_v, sem):
        wid = lax.axis_index("s") * _NC + lax.axis_index("c")
        base = wid * _BPW
        pltpu.sync_copy(idx_hbm.at[pl.ds(base, _BPW)], idx_v)
        cp_m = pltpu.async_copy(means_hbm.at[idx_v], m_v, sem)
        cp_l = pltpu.async_copy(lv_hbm.at[idx_v], lv_v, sem)
        pltpu.sync_copy(eps_hbm.at[pl.ds(base, _BPW), :], e_v)
        cp_m.wait()
        cp_l.wait()

        def row_body(r, carry):
            for c in range(_CHUNKS):
                sl = pl.ds(c * _LANES, _LANES)
                m_v[r, sl] = m_v[r, sl] + jnp.exp(lv_v[r, sl] * 0.5) * e_v[r, sl]
            return carry

        lax.fori_loop(0, _BPW, row_body, 0)
        pltpu.sync_copy(m_v, out_hbm.at[pl.ds(base, _BPW), :])

    return body(means_flat, lv_flat, flat_idx, eps)


_NOISE_CACHE = []


def _noise():
    if not _NOISE_CACHE:
        def build():
            g = jax.random.gumbel(jax.random.key(42), (_B, _K), jnp.float32)
            eps = jax.random.normal(jax.random.key(43), (_B, _D), jnp.float32)
            return g.T, eps
        _NOISE_CACHE.append(jax.jit(build)())
    return _NOISE_CACHE[0]


def kernel(z_means, z_log_vars, z_pis):
    g_t, eps = _noise()
    flat_idx = _select_flat_idx(z_pis.T, g_t)
    means_flat = z_means.reshape(_B * _K, _D)
    lv_flat = z_log_vars.reshape(_B * _K, _D)
    return _sc_gather_reparam(means_flat, lv_flat, flat_idx, eps)


# speedup vs baseline: 1.2891x; 1.0073x over previous
import functools

import jax
import jax.numpy as jnp
from jax import lax
from jax.experimental import pallas as pl
from jax.experimental.pallas import tpu as pltpu
from jax.experimental.pallas import tpu_sc as plsc

_B, _K, _D = 4096, 64, 128
_NC, _NS = 2, 16
_NW = _NC * _NS
_BPW = _B // _NW
_LANES = 16
_CHUNKS = _D // _LANES


def _select_body(pis_ref, g_ref, out_ref):
    x = pis_ref[...]
    g = g_ref[...]
    m = jnp.max(x, axis=0, keepdims=True)
    unnorm = jnp.exp(x - m)
    pis = unnorm / jnp.sum(unnorm, axis=0, keepdims=True)
    s = g + jnp.log(pis)
    smax = jnp.max(s, axis=0, keepdims=True)
    kk = lax.broadcasted_iota(jnp.int32, s.shape, 0)
    bb = lax.broadcasted_iota(jnp.int32, s.shape, 1)
    cand = jnp.where(s == smax, bb * _K + kk, jnp.int32(2**30))
    out_ref[...] = jnp.min(cand, axis=0)


def _select_flat_idx(z_pis_t, g_t):
    return pl.pallas_call(
        _select_body,
        out_shape=jax.ShapeDtypeStruct((_B,), jnp.int32),
    )(z_pis_t, g_t)


def _sc_gather_reparam(means_flat, lv_flat, flat_idx, eps):
    mesh = plsc.VectorSubcoreMesh(core_axis_name="c", subcore_axis_name="s")

    @functools.partial(
        pl.kernel,
        mesh=mesh,
        out_type=jax.ShapeDtypeStruct((_B, _D), jnp.float32),
        scratch_types=[
            pltpu.VMEM((_BPW,), jnp.int32),
            pltpu.VMEM((_BPW, _D), jnp.float32),
            pltpu.VMEM((_BPW, _D), jnp.float32),
            pltpu.VMEM((_BPW, _D), jnp.float32),
            pltpu.SemaphoreType.DMA,
            pltpu.SemaphoreType.DMA,
            pltpu.SemaphoreType.DMA,
        ],
    )
    def body(means_hbm, lv_hbm, idx_hbm, eps_hbm, out_hbm,
             idx_v, m_v, lv_v, e_v, sem_e, sem0, sem1):
        wid = lax.axis_index("s") * _NC + lax.axis_index("c")
        base = wid * _BPW
        half = _BPW // 2
        cp_e = pltpu.async_copy(eps_hbm.at[pl.ds(base, _BPW), :], e_v, sem_e)
        pltpu.sync_copy(idx_hbm.at[pl.ds(base, _BPW)], idx_v)
        cp_m = pltpu.async_copy(means_hbm.at[idx_v], m_v, sem0)
        cp_l = pltpu.async_copy(lv_hbm.at[idx_v], lv_v, sem1)
        cp_m.wait()
        cp_l.wait()
        cp_e.wait()

        def row_body(r, carry):
            for c in range(_CHUNKS):
                sl = pl.ds(c * _LANES, _LANES)
                m_v[r, sl] = m_v[r, sl] + jnp.exp(lv_v[r, sl] * 0.5) * e_v[r, sl]
            return carry

        lax.fori_loop(0, _BPW, row_body, 0)
        pltpu.sync_copy(m_v, out_hbm.at[pl.ds(base, _BPW), :])

    return body(means_flat, lv_flat, flat_idx, eps)


_NOISE_CACHE = []


def _noise():
    if not _NOISE_CACHE:
        def build():
            g = jax.random.gumbel(jax.random.key(42), (_B, _K), jnp.float32)
            eps = jax.random.normal(jax.random.key(43), (_B, _D), jnp.float32)
            return g.T, eps
        _NOISE_CACHE.append(jax.jit(build)())
    return _NOISE_CACHE[0]


def kernel(z_means, z_log_vars, z_pis):
    g_t, eps = _noise()
    flat_idx = _select_flat_idx(z_pis.T, g_t)
    means_flat = z_means.reshape(_B * _K, _D)
    lv_flat = z_log_vars.reshape(_B * _K, _D)
    return _sc_gather_reparam(means_flat, lv_flat, flat_idx, eps)


# speedup vs baseline: 1.2901x; 1.0008x over previous
import functools

import jax
import jax.numpy as jnp
from jax import lax
from jax.experimental import pallas as pl
from jax.experimental.pallas import tpu as pltpu
from jax.experimental.pallas import tpu_sc as plsc

_B, _K, _D = 4096, 64, 128
_NC, _NS = 2, 16
_NW = _NC * _NS
_BPW = _B // _NW
_LANES = 16
_CHUNKS = _D // _LANES


def _select_body(pis_ref, g_ref, out_ref):
    x = pis_ref[...]
    g = g_ref[...]
    m = jnp.max(x, axis=0, keepdims=True)
    unnorm = jnp.exp(x - m)
    pis = unnorm / jnp.sum(unnorm, axis=0, keepdims=True)
    s = g + jnp.log(pis)
    smax = jnp.max(s, axis=0, keepdims=True)
    kk = lax.broadcasted_iota(jnp.int32, s.shape, 0)
    bb = lax.broadcasted_iota(jnp.int32, s.shape, 1)
    cand = jnp.where(s == smax, bb * _K + kk, jnp.int32(2**30))
    out_ref[...] = jnp.min(cand, axis=0)


def _select_flat_idx(z_pis_t, g_t):
    return pl.pallas_call(
        _select_body,
        out_shape=jax.ShapeDtypeStruct((_B,), jnp.int32),
    )(z_pis_t, g_t)


def _sc_gather_reparam(means_flat, lv_flat, flat_idx, eps):
    mesh = plsc.VectorSubcoreMesh(core_axis_name="c", subcore_axis_name="s")

    @functools.partial(
        pl.kernel,
        mesh=mesh,
        out_type=jax.ShapeDtypeStruct((_B, _D), jnp.float32),
        scratch_types=[
            pltpu.VMEM((_BPW,), jnp.int32),
            pltpu.VMEM((_BPW, _D), jnp.float32),
            pltpu.VMEM((_BPW, _D), jnp.float32),
            pltpu.VMEM((_BPW, _D), jnp.float32),
            pltpu.SemaphoreType.DMA,
            pltpu.SemaphoreType.DMA,
            pltpu.SemaphoreType.DMA,
        ],
    )
    def body(means_hbm, lv_hbm, idx_hbm, eps_hbm, out_hbm,
             idx_v, m_v, lv_v, e_v, sem_e, sem0, sem1):
        wid = lax.axis_index("s") * _NC + lax.axis_index("c")
        base = wid * _BPW
        half = _BPW // 2
        cp_e = pltpu.async_copy(eps_hbm.at[pl.ds(base, _BPW), :], e_v, sem_e)
        pltpu.sync_copy(idx_hbm.at[pl.ds(base, _BPW)], idx_v)
        cp_m = pltpu.async_copy(means_hbm.at[idx_v], m_v, sem0)
        cp_l = pltpu.async_copy(lv_hbm.at[idx_v], lv_v, sem1)
        cp_m.wait()
        cp_l.wait()
        cp_e.wait()

        def row_body(i, carry):
            r = i * 2
            for rr in range(2):
                for c in range(_CHUNKS):
                    sl = pl.ds(c * _LANES, _LANES)
                    m_v[r + rr, sl] = (m_v[r + rr, sl]
                                       + jnp.exp(lv_v[r + rr, sl] * 0.5) * e_v[r + rr, sl])
            return carry

        lax.fori_loop(0, _BPW // 2, row_body, 0)
        pltpu.sync_copy(m_v, out_hbm.at[pl.ds(base, _BPW), :])

    return body(means_flat, lv_flat, flat_idx, eps)


_NOISE_CACHE = []


def _noise():
    if not _NOISE_CACHE:
        def build():
            g = jax.random.gumbel(jax.random.key(42), (_B, _K), jnp.float32)
            eps = jax.random.normal(jax.random.key(43), (_B, _D), jnp.float32)
            return g.T, eps
        _NOISE_CACHE.append(jax.jit(build)())
    return _NOISE_CACHE[0]


def kernel(z_means, z_log_vars, z_pis):
    g_t, eps = _noise()
    flat_idx = _select_flat_idx(z_pis.T, g_t)
    means_flat = z_means.reshape(_B * _K, _D)
    lv_flat = z_log_vars.reshape(_B * _K, _D)
    return _sc_gather_reparam(means_flat, lv_flat, flat_idx, eps)
